# baseline (speedup 1.0000x reference)
import functools

import jax
import jax.numpy as jnp
from jax import lax
from jax.experimental import pallas as pl
from jax.experimental.pallas import tpu as pltpu
from jax.experimental.pallas import tpu_sc as plsc

NUM_EXPERTS = 8
N_EMBD = 1024
D_FFN = 2048
BLK = 64
NB = 80
NR = NB * BLK
NACT = 80
L = 16
CH = 32
NCH = NR // CH
REPS = NCH // 32
PB = 128


def _router_body(x_ref, wr_ref, sel_ref, wn_ref):
    x = x_ref[...]
    wr = wr_ref[...]
    logits = lax.dot_general(
        x, wr, (((1,), (1,)), ((), ())), preferred_element_type=jnp.float32
    )
    m = jnp.max(logits, axis=-1, keepdims=True)
    ex = jnp.exp(logits - m)
    probs = ex / jnp.sum(ex, axis=-1, keepdims=True)
    e_iota = lax.broadcasted_iota(jnp.int32, probs.shape, 1)
    m1 = jnp.max(probs, axis=-1, keepdims=True)
    i1 = jnp.min(jnp.where(probs == m1, e_iota, NUM_EXPERTS), axis=-1, keepdims=True)
    masked = jnp.where(e_iota == i1, -jnp.inf, probs)
    m2 = jnp.max(masked, axis=-1, keepdims=True)
    i2 = jnp.min(jnp.where(masked == m2, e_iota, NUM_EXPERTS), axis=-1, keepdims=True)
    s = m1 + m2
    sel_ref[...] = jnp.concatenate([i1, i2], axis=1)
    wn_ref[...] = jnp.concatenate([m1 / s, m2 / s], axis=1)


def _index_body(sel_hbm, wn_hbm, slots_hbm, wvec_hbm, meta_hbm, posp_hbm,
                selv, wnv, slotsv, wvecv, metav, posv, psem):
    nslots = sel_hbm.shape[0]
    nv = nslots // L
    wid = lax.axis_index("s") * 2 + lax.axis_index("c")
    iota = lax.iota(jnp.int32, L)

    @pl.when(wid < NUM_EXPERTS + 1)
    def _():
        pltpu.sync_copy(sel_hbm, selv)

        def count_step(j, acc):
            v = selv[pl.ds(j * L, L)]
            return tuple(
                acc[f] + jnp.where(v == f, 1, 0) for f in range(NUM_EXPERTS)
            )

        zero = jnp.zeros((L,), jnp.int32)
        acc = lax.fori_loop(0, nv, count_step, (zero,) * NUM_EXPERTS)
        counts = [jnp.sum(a) for a in acc]
        bcs = [(c + BLK - 1) // BLK for c in counts]

        @pl.when(wid < NUM_EXPERTS)
        def _():
            pltpu.sync_copy(wn_hbm, wnv)
            e = wid
            base = jnp.int32(0)
            for f in range(NUM_EXPERTS):
                base = base + jnp.where(f < e, bcs[f], 0)
            base = base * BLK
            my_bc = jnp.int32(0)
            for f in range(NUM_EXPERTS):
                my_bc = my_bc + jnp.where(f == e, bcs[f], 0)

            def fill_step(j, carry):
                off = base + j * L
                slotsv[pl.ds(off, L)] = nslots + ((off + iota) & 255)
                wvecv[pl.ds(off, L)] = jnp.zeros((L,), jnp.float32)
                return carry

            lax.fori_loop(0, my_bc * (BLK // L), fill_step, 0)

            def zero_step(j, carry):
                posv[pl.ds(j * L, L)] = jnp.zeros((L,), jnp.int32)
                return carry

            lax.fori_loop(0, nslots // L, zero_step, 0)

            tt = nslots // 2

            def rank_step(j, cnt):
                sl = j * L + iota
                v = selv[pl.ds(j * L, L)]
                mk = v == e
                c = plsc.cumsum(jnp.where(mk, 1, 0))
                p = base + cnt + c - 1
                plsc.store_scatter(slotsv, [p], sl, mask=mk)
                wvals = wnv[pl.ds(j * L, L)]
                plsc.store_scatter(wvecv, [p], wvals, mask=mk)
                td = (sl & 1) * tt + lax.shift_right_logical(sl, 1)
                plsc.store_scatter(posv, [td], p + 1, mask=mk)
                return cnt + jnp.where(mk, 1, 0).sum()

            lax.fori_loop(0, nv, rank_step, jnp.int32(0))

            maxbc = nslots // BLK
            for j in range(maxbc):
                @pl.when(j < my_bc)
                def _(j=j):
                    off = base + j * BLK
                    pltpu.async_copy(slotsv.at[pl.ds(off, BLK)],
                                     slots_hbm.at[pl.ds(off, BLK)], psem)
                    pltpu.async_copy(wvecv.at[pl.ds(off, BLK)],
                                     wvec_hbm.at[pl.ds(off, BLK)], psem)

            pltpu.sync_copy(posv, posp_hbm.at[e])

            def drain(j, carry):
                pltpu.make_async_copy(
                    slots_hbm.at[pl.ds(0, BLK)],
                    slotsv.at[pl.ds(0, BLK)], psem).wait()
                return carry

            lax.fori_loop(0, my_bc * 2, drain, 0)

        @pl.when(wid == NUM_EXPERTS)
        def _():
            prefix = []
            run = jnp.int32(0)
            for f in range(NUM_EXPERTS):
                run = run + bcs[f]
                prefix.append(run)
            nact = prefix[-1]
            for v in range(NACT // L + 1):
                blk = v * L + iota
                eid = jnp.zeros((L,), jnp.int32)
                for f in range(NUM_EXPERTS - 1):
                    eid = eid + jnp.where(blk >= prefix[f], 1, 0)
                eid = jnp.where(blk < nact, eid, 0)
                if v == NACT // L:
                    eid = jnp.where(iota == 0, nact, 0)
                metav[pl.ds(v * L, L)] = eid
            pltpu.sync_copy(metav, meta_hbm)


def _spread_body(xf_hbm, posp_hbm, xs_hbm,
                 ppa, ppb, pva, pvb, rb0, rb1, psem, sem0, sem1,
                 os0, os1, os2, os3):
    t_tokens = xf_hbm.shape[0]
    wid = lax.axis_index("s") * 2 + lax.axis_index("c")
    rbs = (rb0, rb1)
    pps = (ppa, ppb)
    pvs = (pva, pvb)
    sems = (sem0, sem1)
    osems = ((os0, os1), (os2, os3))

    cps = [None, None]
    for r in range(2):
        g = wid + 32 * r
        for e in range(NUM_EXPERTS):
            pltpu.async_copy(posp_hbm.at[e, pl.ds(g * 32, 32)],
                             pps[r].at[e, pl.ds(0, 32)], psem)
            pltpu.async_copy(posp_hbm.at[e, pl.ds(t_tokens + g * 32, 32)],
                             pps[r].at[e, pl.ds(32, 32)], psem)
        cps[r] = pltpu.async_copy(
            xf_hbm.at[pl.ds(g * 32, 32)], rbs[r], sems[r])

    def drain(j, carry):
        pltpu.make_async_copy(
            posp_hbm.at[0, pl.ds(0, 32)],
            ppa.at[0, pl.ds(0, 32)], psem).wait()
        return carry

    lax.fori_loop(0, 2 * 2 * NUM_EXPERTS, drain, 0)

    for r in range(2):
        for q in range(4):
            acc = pps[r][0, pl.ds(q * L, L)]
            for e in range(1, NUM_EXPERTS):
                acc = jnp.maximum(acc, pps[r][e, pl.ds(q * L, L)])
            pvs[r][q // 2, pl.ds((q % 2) * L, L)] = acc - 1

    ops = []
    for r in range(2):
        cps[r].wait()
        ops.append(pltpu.async_copy(
            rbs[r], xs_hbm.at[pvs[r].at[0]], osems[r][0]))
        ops.append(pltpu.async_copy(
            rbs[r], xs_hbm.at[pvs[r].at[1]], osems[r][1]))
    for o in ops:
        o.wait()


def _mm_body(meta_ref, xs_ref, w1_ref, w2_ref, wv_ref, out_ref):
    b = pl.program_id(0)
    nact = meta_ref[NACT]

    @pl.when(b < nact)
    def _():
        h = lax.dot_general(
            xs_ref[0], w1_ref[...], (((1,), (0,)), ((), ())),
            preferred_element_type=jnp.float32,
        )
        y = lax.dot_general(
            h, w2_ref[...], (((1,), (0,)), ((), ())),
            preferred_element_type=jnp.float32,
        )
        out_ref[0] = y * wv_ref[0]


def _scatter_body(ys_hbm, slots_hbm, meta_hbm, buf_hbm,
                  sall, sidx, rb0, rb1, metav, sem0, sem1, osem0, osem1):
    nslots = buf_hbm.shape[0] - 256
    t_tokens = nslots // 2
    wid = lax.axis_index("s") * 2 + lax.axis_index("c")
    iota = lax.iota(jnp.int32, L)
    pltpu.sync_copy(meta_hbm.at[pl.ds(NACT, L)], metav)
    nch = (BLK // CH) * jnp.max(metav[...])
    pltpu.sync_copy(slots_hbm, sall)
    rbs = (rb0, rb1)
    sems = (sem0, sem1)
    osems = (osem0, osem1)

    for r in range(REPS):
        g = wid + 32 * r
        base = g * CH
        for q in range(CH // L):
            s = sall[pl.ds(base + q * L, L)]
            real = jnp.logical_and(s < nslots, g < nch)
            dst = jnp.where(
                real,
                (s & 1) * t_tokens + lax.shift_right_logical(s, 1),
                nslots + ((base + q * L + iota) & 255))
            sidx[r, pl.ds(q * L, L)] = dst
    cps = [None, None]
    ops = [None, None]
    cps[0] = pltpu.async_copy(ys_hbm.at[pl.ds(wid * CH, CH)], rb0, sem0)
    for r in range(REPS):
        if r < REPS - 1:
            if ops[(r + 1) % 2] is not None:
                ops[(r + 1) % 2].wait()
                ops[(r + 1) % 2] = None
            cps[(r + 1) % 2] = pltpu.async_copy(
                ys_hbm.at[pl.ds((wid + 32 * (r + 1)) * CH, CH)],
                rbs[(r + 1) % 2], sems[(r + 1) % 2])
        cps[r % 2].wait()
        ops[r % 2] = pltpu.async_copy(
            rbs[r % 2], buf_hbm.at[sidx.at[r]], osems[r % 2])
    for q in range(2):
        if ops[q] is not None:
            ops[q].wait()


def _pair_body(a_ref, b_ref, out_ref):
    out_ref[...] = a_ref[...] + b_ref[...]


def kernel(x, w_router, w1, w2):
    b, s, d = x.shape
    t = b * s
    nslots = 2 * t
    xf = x.reshape(t, d)

    sel, wn = pl.pallas_call(
        _router_body,
        out_shape=(
            jax.ShapeDtypeStruct((t, 2), jnp.int32),
            jax.ShapeDtypeStruct((t, 2), jnp.float32),
        ),
    )(xf, w_router)

    sel_flat = sel.reshape(nslots)
    wn_flat = wn.reshape(nslots)

    mesh = plsc.VectorSubcoreMesh(core_axis_name="c", subcore_axis_name="s")
    sc_params = pltpu.CompilerParams(needs_layout_passes=False)

    slots, wvec, meta, posp = pl.kernel(
        _index_body,
        out_type=(
            jax.ShapeDtypeStruct((NR,), jnp.int32),
            jax.ShapeDtypeStruct((NR,), jnp.float32),
            jax.ShapeDtypeStruct((NACT + L,), jnp.int32),
            jax.ShapeDtypeStruct((NUM_EXPERTS, nslots), jnp.int32),
        ),
        mesh=mesh,
        scratch_types=[
            pltpu.VMEM((nslots,), jnp.int32),
            pltpu.VMEM((nslots,), jnp.float32),
            pltpu.VMEM((NR,), jnp.int32),
            pltpu.VMEM((NR,), jnp.float32),
            pltpu.VMEM((NACT + L,), jnp.int32),
            pltpu.VMEM((nslots,), jnp.int32),
            pltpu.SemaphoreType.DMA,
        ],
        compiler_params=sc_params,
    )(sel_flat, wn_flat)

    xs = pl.kernel(
        _spread_body,
        out_type=jax.ShapeDtypeStruct((NR, d), jnp.float32),
        mesh=mesh,
        scratch_types=[
            pltpu.VMEM((NUM_EXPERTS, 64), jnp.int32),
            pltpu.VMEM((NUM_EXPERTS, 64), jnp.int32),
            pltpu.VMEM((2, 32), jnp.int32),
            pltpu.VMEM((2, 32), jnp.int32),
            pltpu.VMEM((32, d), jnp.float32),
            pltpu.VMEM((32, d), jnp.float32),
            pltpu.SemaphoreType.DMA,
            pltpu.SemaphoreType.DMA,
            pltpu.SemaphoreType.DMA,
            pltpu.SemaphoreType.DMA,
            pltpu.SemaphoreType.DMA,
            pltpu.SemaphoreType.DMA,
            pltpu.SemaphoreType.DMA,
        ],
        compiler_params=sc_params,
    )(xf, posp)

    xs3 = xs.reshape(NB, BLK, d)
    wv3 = wvec.reshape(NB, BLK, 1)

    ys = pl.pallas_call(
        _mm_body,
        grid_spec=pltpu.PrefetchScalarGridSpec(
            num_scalar_prefetch=1,
            grid=(NB,),
            in_specs=[
                pl.BlockSpec((1, BLK, d), lambda i, m: (i, 0, 0)),
                pl.BlockSpec((d, D_FFN), lambda i, m: (0, m[i])),
                pl.BlockSpec((D_FFN, d), lambda i, m: (m[i], 0)),
                pl.BlockSpec((1, BLK, 1), lambda i, m: (i, 0, 0)),
            ],
            out_specs=pl.BlockSpec((1, BLK, d), lambda i, m: (i, 0, 0)),
        ),
        out_shape=jax.ShapeDtypeStruct((NB, BLK, d), jnp.float32),
        compiler_params=pltpu.CompilerParams(
            dimension_semantics=("arbitrary",),
        ),
    )(meta, xs3, w1, w2, wv3)

    buf = pl.kernel(
        _scatter_body,
        out_type=jax.ShapeDtypeStruct((nslots + 256, d), jnp.float32),
        mesh=mesh,
        scratch_types=[
            pltpu.VMEM((NR,), jnp.int32),
            pltpu.VMEM((REPS, CH), jnp.int32),
            pltpu.VMEM((CH, d), jnp.float32),
            pltpu.VMEM((CH, d), jnp.float32),
            pltpu.VMEM((L,), jnp.int32),
            pltpu.SemaphoreType.DMA,
            pltpu.SemaphoreType.DMA,
            pltpu.SemaphoreType.DMA,
            pltpu.SemaphoreType.DMA,
        ],
        compiler_params=sc_params,
    )(ys.reshape(NR, d), slots, meta)

    nt = t // PB
    out = pl.pallas_call(
        _pair_body,
        grid=(nt,),
        in_specs=[
            pl.BlockSpec((PB, d), lambda i: (i, 0)),
            pl.BlockSpec((PB, d), lambda i: (nt + i, 0)),
        ],
        out_specs=pl.BlockSpec((PB, d), lambda i: (i, 0)),
        out_shape=jax.ShapeDtypeStruct((t, d), jnp.float32),
    )(buf, buf)

    return out.reshape(b, s, d)


# speedup vs baseline: 1.3340x; 1.3340x over previous
import functools

import jax
import jax.numpy as jnp
from jax import lax
from jax.experimental import pallas as pl
from jax.experimental.pallas import tpu as pltpu
from jax.experimental.pallas import tpu_sc as plsc

NUM_EXPERTS = 8
N_EMBD = 1024
D_FFN = 2048
BLK = 128
NB = 40
NR = NB * BLK
L = 16
CH = 32
NCH = NR // CH
REPS = NCH // 32


def _router_body(x_ref, wr_ref, sel_ref, wn_ref):
    x = x_ref[...]
    wr = wr_ref[...]
    logits = lax.dot_general(
        x, wr, (((1,), (1,)), ((), ())), preferred_element_type=jnp.float32
    )
    m = jnp.max(logits, axis=-1, keepdims=True)
    ex = jnp.exp(logits - m)
    probs = ex / jnp.sum(ex, axis=-1, keepdims=True)
    e_iota = lax.broadcasted_iota(jnp.int32, probs.shape, 1)
    m1 = jnp.max(probs, axis=-1, keepdims=True)
    i1 = jnp.min(jnp.where(probs == m1, e_iota, NUM_EXPERTS), axis=-1, keepdims=True)
    masked = jnp.where(e_iota == i1, -jnp.inf, probs)
    m2 = jnp.max(masked, axis=-1, keepdims=True)
    i2 = jnp.min(jnp.where(masked == m2, e_iota, NUM_EXPERTS), axis=-1, keepdims=True)
    s = m1 + m2
    sel_ref[...] = jnp.concatenate([i1, i2], axis=1)
    wn_ref[...] = jnp.concatenate([m1 / s, m2 / s], axis=1)


def _index_body(sel_hbm, wn_hbm, slots_hbm, wvec_hbm, meta_hbm, posp_hbm,
                selv, wnv, slotsv, wvecv, metav, posv, psem):
    nslots = sel_hbm.shape[0]
    nv = nslots // L
    wid = lax.axis_index("s") * 2 + lax.axis_index("c")
    iota = lax.iota(jnp.int32, L)

    @pl.when(wid < NUM_EXPERTS + 1)
    def _():
        pltpu.sync_copy(sel_hbm, selv)

        def count_step(j, acc):
            v = selv[pl.ds(j * L, L)]
            return tuple(
                acc[f] + jnp.where(v == f, 1, 0) for f in range(NUM_EXPERTS)
            )

        zero = jnp.zeros((L,), jnp.int32)
        acc = lax.fori_loop(0, nv, count_step, (zero,) * NUM_EXPERTS)
        counts = [jnp.sum(a) for a in acc]
        bcs = [(c + BLK - 1) // BLK for c in counts]

        @pl.when(wid < NUM_EXPERTS)
        def _():
            pltpu.sync_copy(wn_hbm, wnv)
            e = wid
            base = jnp.int32(0)
            for f in range(NUM_EXPERTS):
                base = base + jnp.where(f < e, bcs[f], 0)
            base = base * BLK
            my_bc = jnp.int32(0)
            for f in range(NUM_EXPERTS):
                my_bc = my_bc + jnp.where(f == e, bcs[f], 0)

            def fill_step(j, carry):
                off = base + j * L
                slotsv[pl.ds(off, L)] = nslots + ((off + iota) & 255)
                wvecv[pl.ds(off, L)] = jnp.zeros((L,), jnp.float32)
                return carry

            lax.fori_loop(0, my_bc * (BLK // L), fill_step, 0)

            def zero_step(j, carry):
                posv[pl.ds(j * L, L)] = jnp.zeros((L,), jnp.int32)
                return carry

            lax.fori_loop(0, nslots // L, zero_step, 0)

            tt = nslots // 2

            def rank_step(j, cnt):
                sl = j * L + iota
                v = selv[pl.ds(j * L, L)]
                mk = v == e
                c = plsc.cumsum(jnp.where(mk, 1, 0))
                p = base + cnt + c - 1
                plsc.store_scatter(slotsv, [p], sl, mask=mk)
                wvals = wnv[pl.ds(j * L, L)]
                plsc.store_scatter(wvecv, [p], wvals, mask=mk)
                td = (sl & 1) * tt + lax.shift_right_logical(sl, 1)
                plsc.store_scatter(posv, [td], p + 1, mask=mk)
                return cnt + jnp.where(mk, 1, 0).sum()

            lax.fori_loop(0, nv, rank_step, jnp.int32(0))

            maxbc = nslots // BLK
            for j in range(maxbc):
                @pl.when(j < my_bc)
                def _(j=j):
                    off = base + j * BLK
                    pltpu.async_copy(slotsv.at[pl.ds(off, BLK)],
                                     slots_hbm.at[pl.ds(off, BLK)], psem)
                    pltpu.async_copy(wvecv.at[pl.ds(off, BLK)],
                                     wvec_hbm.at[pl.ds(off, BLK)], psem)

            pltpu.sync_copy(posv, posp_hbm.at[e])

            def drain(j, carry):
                pltpu.make_async_copy(
                    slots_hbm.at[pl.ds(0, BLK)],
                    slotsv.at[pl.ds(0, BLK)], psem).wait()
                return carry

            lax.fori_loop(0, my_bc * 2, drain, 0)

        @pl.when(wid == NUM_EXPERTS)
        def _():
            prefix = []
            run = jnp.int32(0)
            for f in range(NUM_EXPERTS):
                run = run + bcs[f]
                prefix.append(run)
            nact = prefix[-1]
            for v in range(4):
                blk = v * L + iota
                eid = jnp.zeros((L,), jnp.int32)
                for f in range(NUM_EXPERTS - 1):
                    eid = eid + jnp.where(blk >= prefix[f], 1, 0)
                eid = jnp.where(blk < nact, eid, 0)
                if v == 3:
                    eid = jnp.where(iota == 0, nact, 0)
                metav[pl.ds(v * L, L)] = eid
            pltpu.sync_copy(metav, meta_hbm)


def _spread_body(xf_hbm, posp_hbm, xs_hbm,
                 ppa, ppb, pva, pvb, rb0, rb1, psem, sem0, sem1,
                 os0, os1, os2, os3):
    t_tokens = xf_hbm.shape[0]
    wid = lax.axis_index("s") * 2 + lax.axis_index("c")
    rbs = (rb0, rb1)
    pps = (ppa, ppb)
    pvs = (pva, pvb)
    sems = (sem0, sem1)
    osems = ((os0, os1), (os2, os3))

    cps = [None, None]
    for r in range(2):
        g = wid + 32 * r
        for e in range(NUM_EXPERTS):
            pltpu.async_copy(posp_hbm.at[e, pl.ds(g * 32, 32)],
                             pps[r].at[e, pl.ds(0, 32)], psem)
            pltpu.async_copy(posp_hbm.at[e, pl.ds(t_tokens + g * 32, 32)],
                             pps[r].at[e, pl.ds(32, 32)], psem)
        cps[r] = pltpu.async_copy(
            xf_hbm.at[pl.ds(g * 32, 32)], rbs[r], sems[r])

    def drain(j, carry):
        pltpu.make_async_copy(
            posp_hbm.at[0, pl.ds(0, 32)],
            ppa.at[0, pl.ds(0, 32)], psem).wait()
        return carry

    lax.fori_loop(0, 2 * 2 * NUM_EXPERTS, drain, 0)

    for r in range(2):
        for q in range(4):
            acc = pps[r][0, pl.ds(q * L, L)]
            for e in range(1, NUM_EXPERTS):
                acc = jnp.maximum(acc, pps[r][e, pl.ds(q * L, L)])
            pvs[r][q // 2, pl.ds((q % 2) * L, L)] = acc - 1

    ops = []
    for r in range(2):
        cps[r].wait()
        ops.append(pltpu.async_copy(
            rbs[r], xs_hbm.at[pvs[r].at[0]], osems[r][0]))
        ops.append(pltpu.async_copy(
            rbs[r], xs_hbm.at[pvs[r].at[1]], osems[r][1]))
    for o in ops:
        o.wait()


def _mm_body(meta_ref, xs_ref, w1_ref, w2_ref, wv_ref, out_ref):
    b = pl.program_id(0)
    nact = meta_ref[48]

    @pl.when(b < nact)
    def _():
        h = lax.dot_general(
            xs_ref[0], w1_ref[...], (((1,), (0,)), ((), ())),
            preferred_element_type=jnp.float32,
        )
        y = lax.dot_general(
            h, w2_ref[...], (((1,), (0,)), ((), ())),
            preferred_element_type=jnp.float32,
        )
        out_ref[0] = y * wv_ref[0]


def _scatter_body(ys_hbm, slots_hbm, meta_hbm, buf_hbm,
                  sall, sidx, rb0, rb1, metav, sem0, sem1, osem0, osem1):
    nslots = buf_hbm.shape[0] - 256
    t_tokens = nslots // 2
    wid = lax.axis_index("s") * 2 + lax.axis_index("c")
    iota = lax.iota(jnp.int32, L)
    pltpu.sync_copy(meta_hbm.at[pl.ds(48, L)], metav)
    nch = 4 * jnp.max(metav[...])
    pltpu.sync_copy(slots_hbm, sall)
    rbs = (rb0, rb1)
    sems = (sem0, sem1)
    osems = (osem0, osem1)

    for r in range(REPS):
        g = wid + 32 * r
        base = g * CH
        for q in range(CH // L):
            s = sall[pl.ds(base + q * L, L)]
            real = jnp.logical_and(s < nslots, g < nch)
            dst = jnp.where(
                real,
                (s & 1) * t_tokens + lax.shift_right_logical(s, 1),
                nslots + ((base + q * L + iota) & 255))
            sidx[r, pl.ds(q * L, L)] = dst
    cps = [None, None]
    ops = [None, None]
    cps[0] = pltpu.async_copy(ys_hbm.at[pl.ds(wid * CH, CH)], rb0, sem0)
    for r in range(REPS):
        if r < REPS - 1:
            if ops[(r + 1) % 2] is not None:
                ops[(r + 1) % 2].wait()
                ops[(r + 1) % 2] = None
            cps[(r + 1) % 2] = pltpu.async_copy(
                ys_hbm.at[pl.ds((wid + 32 * (r + 1)) * CH, CH)],
                rbs[(r + 1) % 2], sems[(r + 1) % 2])
        cps[r % 2].wait()
        ops[r % 2] = pltpu.async_copy(
            rbs[r % 2], buf_hbm.at[sidx.at[r]], osems[r % 2])
    for q in range(2):
        if ops[q] is not None:
            ops[q].wait()


def _pair_body(a_ref, b_ref, out_ref):
    out_ref[...] = a_ref[...] + b_ref[...]


def kernel(x, w_router, w1, w2):
    b, s, d = x.shape
    t = b * s
    nslots = 2 * t
    xf = x.reshape(t, d)

    sel, wn = pl.pallas_call(
        _router_body,
        out_shape=(
            jax.ShapeDtypeStruct((t, 2), jnp.int32),
            jax.ShapeDtypeStruct((t, 2), jnp.float32),
        ),
    )(xf, w_router)

    sel_flat = sel.reshape(nslots)
    wn_flat = wn.reshape(nslots)

    mesh = plsc.VectorSubcoreMesh(core_axis_name="c", subcore_axis_name="s")
    sc_params = pltpu.CompilerParams(needs_layout_passes=False)

    slots, wvec, meta, posp = pl.kernel(
        _index_body,
        out_type=(
            jax.ShapeDtypeStruct((NR,), jnp.int32),
            jax.ShapeDtypeStruct((NR,), jnp.float32),
            jax.ShapeDtypeStruct((64,), jnp.int32),
            jax.ShapeDtypeStruct((NUM_EXPERTS, nslots), jnp.int32),
        ),
        mesh=mesh,
        scratch_types=[
            pltpu.VMEM((nslots,), jnp.int32),
            pltpu.VMEM((nslots,), jnp.float32),
            pltpu.VMEM((NR,), jnp.int32),
            pltpu.VMEM((NR,), jnp.float32),
            pltpu.VMEM((64,), jnp.int32),
            pltpu.VMEM((nslots,), jnp.int32),
            pltpu.SemaphoreType.DMA,
        ],
        compiler_params=sc_params,
    )(sel_flat, wn_flat)

    xs = pl.kernel(
        _spread_body,
        out_type=jax.ShapeDtypeStruct((NR, d), jnp.float32),
        mesh=mesh,
        scratch_types=[
            pltpu.VMEM((NUM_EXPERTS, 64), jnp.int32),
            pltpu.VMEM((NUM_EXPERTS, 64), jnp.int32),
            pltpu.VMEM((2, 32), jnp.int32),
            pltpu.VMEM((2, 32), jnp.int32),
            pltpu.VMEM((32, d), jnp.float32),
            pltpu.VMEM((32, d), jnp.float32),
            pltpu.SemaphoreType.DMA,
            pltpu.SemaphoreType.DMA,
            pltpu.SemaphoreType.DMA,
            pltpu.SemaphoreType.DMA,
            pltpu.SemaphoreType.DMA,
            pltpu.SemaphoreType.DMA,
            pltpu.SemaphoreType.DMA,
        ],
        compiler_params=sc_params,
    )(xf, posp)

    xs3 = xs.reshape(NB, BLK, d)
    wv3 = wvec.reshape(NB, BLK, 1)

    ys = pl.pallas_call(
        _mm_body,
        grid_spec=pltpu.PrefetchScalarGridSpec(
            num_scalar_prefetch=1,
            grid=(NB,),
            in_specs=[
                pl.BlockSpec((1, BLK, d), lambda i, m: (i, 0, 0)),
                pl.BlockSpec((d, D_FFN), lambda i, m: (0, m[i])),
                pl.BlockSpec((D_FFN, d), lambda i, m: (m[i], 0)),
                pl.BlockSpec((1, BLK, 1), lambda i, m: (i, 0, 0)),
            ],
            out_specs=pl.BlockSpec((1, BLK, d), lambda i, m: (i, 0, 0)),
        ),
        out_shape=jax.ShapeDtypeStruct((NB, BLK, d), jnp.float32),
        compiler_params=pltpu.CompilerParams(
            dimension_semantics=("arbitrary",),
        ),
    )(meta, xs3, w1, w2, wv3)

    buf = pl.kernel(
        _scatter_body,
        out_type=jax.ShapeDtypeStruct((nslots + 256, d), jnp.float32),
        mesh=mesh,
        scratch_types=[
            pltpu.VMEM((NR,), jnp.int32),
            pltpu.VMEM((REPS, CH), jnp.int32),
            pltpu.VMEM((CH, d), jnp.float32),
            pltpu.VMEM((CH, d), jnp.float32),
            pltpu.VMEM((L,), jnp.int32),
            pltpu.SemaphoreType.DMA,
            pltpu.SemaphoreType.DMA,
            pltpu.SemaphoreType.DMA,
            pltpu.SemaphoreType.DMA,
        ],
        compiler_params=sc_params,
    )(ys.reshape(NR, d), slots, meta)

    nt = t // BLK
    out = pl.pallas_call(
        _pair_body,
        grid=(nt,),
        in_specs=[
            pl.BlockSpec((BLK, d), lambda i: (i, 0)),
            pl.BlockSpec((BLK, d), lambda i: (nt + i, 0)),
        ],
        out_specs=pl.BlockSpec((BLK, d), lambda i: (i, 0)),
        out_shape=jax.ShapeDtypeStruct((t, d), jnp.float32),
    )(buf, buf)

    return out.reshape(b, s, d)


# speedup vs baseline: 1.3859x; 1.0389x over previous
import functools

import jax
import jax.numpy as jnp
from jax import lax
from jax.experimental import pallas as pl
from jax.experimental.pallas import tpu as pltpu
from jax.experimental.pallas import tpu_sc as plsc

NUM_EXPERTS = 8
N_EMBD = 1024
D_FFN = 2048
BLK = 128
NB = 40
NR = NB * BLK
L = 16
CH = 32
NCH = NR // CH
REPS = NCH // 32


def _router_body(x_ref, wr_ref, sel_ref, wn_ref):
    x = x_ref[...]
    wr = wr_ref[...]
    logits = lax.dot_general(
        x, wr, (((1,), (1,)), ((), ())), preferred_element_type=jnp.float32
    )
    m = jnp.max(logits, axis=-1, keepdims=True)
    ex = jnp.exp(logits - m)
    probs = ex / jnp.sum(ex, axis=-1, keepdims=True)
    e_iota = lax.broadcasted_iota(jnp.int32, probs.shape, 1)
    m1 = jnp.max(probs, axis=-1, keepdims=True)
    i1 = jnp.min(jnp.where(probs == m1, e_iota, NUM_EXPERTS), axis=-1, keepdims=True)
    masked = jnp.where(e_iota == i1, -jnp.inf, probs)
    m2 = jnp.max(masked, axis=-1, keepdims=True)
    i2 = jnp.min(jnp.where(masked == m2, e_iota, NUM_EXPERTS), axis=-1, keepdims=True)
    s = m1 + m2
    sel_ref[...] = jnp.concatenate([i1, i2], axis=1)
    wn_ref[...] = jnp.concatenate([m1 / s, m2 / s], axis=1)


def _index_body(sel_hbm, wn_hbm, slots_hbm, wvec_hbm, meta_hbm, posp_hbm,
                selv, wnv, slotsv, wvecv, metav, posv, psem):
    nslots = sel_hbm.shape[0]
    nv = nslots // L
    wid = lax.axis_index("s") * 2 + lax.axis_index("c")
    iota = lax.iota(jnp.int32, L)

    @pl.when(wid < NUM_EXPERTS + 1)
    def _():
        pltpu.sync_copy(sel_hbm, selv)

        def count_step(j, acc):
            v = selv[pl.ds(j * L, L)]
            return tuple(
                acc[f] + jnp.where(v == f, 1, 0) for f in range(NUM_EXPERTS)
            )

        zero = jnp.zeros((L,), jnp.int32)
        acc = lax.fori_loop(0, nv, count_step, (zero,) * NUM_EXPERTS)
        counts = [jnp.sum(a) for a in acc]
        bcs = [(c + BLK - 1) // BLK for c in counts]

        @pl.when(wid < NUM_EXPERTS)
        def _():
            pltpu.sync_copy(wn_hbm, wnv)
            e = wid
            base = jnp.int32(0)
            for f in range(NUM_EXPERTS):
                base = base + jnp.where(f < e, bcs[f], 0)
            base = base * BLK
            my_bc = jnp.int32(0)
            for f in range(NUM_EXPERTS):
                my_bc = my_bc + jnp.where(f == e, bcs[f], 0)

            def fill_step(j, carry):
                off = base + j * L
                slotsv[pl.ds(off, L)] = nslots + ((off + iota) & 255)
                wvecv[pl.ds(off, L)] = jnp.zeros((L,), jnp.float32)
                return carry

            lax.fori_loop(0, my_bc * (BLK // L), fill_step, 0)

            def zero_step(j, carry):
                posv[pl.ds(j * L, L)] = jnp.zeros((L,), jnp.int32)
                return carry

            lax.fori_loop(0, nslots // L, zero_step, 0)

            tt = nslots // 2

            def rank_step(j, cnt):
                sl = j * L + iota
                v = selv[pl.ds(j * L, L)]
                mk = v == e
                c = plsc.cumsum(jnp.where(mk, 1, 0))
                p = base + cnt + c - 1
                plsc.store_scatter(slotsv, [p], sl, mask=mk)
                wvals = wnv[pl.ds(j * L, L)]
                plsc.store_scatter(wvecv, [p], wvals, mask=mk)
                td = (sl & 1) * tt + lax.shift_right_logical(sl, 1)
                plsc.store_scatter(posv, [td], p + 1, mask=mk)
                return cnt + jnp.where(mk, 1, 0).sum()

            lax.fori_loop(0, nv, rank_step, jnp.int32(0))

            maxbc = nslots // BLK
            for j in range(maxbc):
                @pl.when(j < my_bc)
                def _(j=j):
                    off = base + j * BLK
                    pltpu.async_copy(slotsv.at[pl.ds(off, BLK)],
                                     slots_hbm.at[pl.ds(off, BLK)], psem)
                    pltpu.async_copy(wvecv.at[pl.ds(off, BLK)],
                                     wvec_hbm.at[pl.ds(off, BLK)], psem)

            pltpu.sync_copy(posv, posp_hbm.at[e])

            def drain(j, carry):
                pltpu.make_async_copy(
                    slots_hbm.at[pl.ds(0, BLK)],
                    slotsv.at[pl.ds(0, BLK)], psem).wait()
                return carry

            lax.fori_loop(0, my_bc * 2, drain, 0)

        @pl.when(wid == NUM_EXPERTS)
        def _():
            prefix = []
            run = jnp.int32(0)
            for f in range(NUM_EXPERTS):
                run = run + bcs[f]
                prefix.append(run)
            nact = prefix[-1]
            for v in range(4):
                blk = v * L + iota
                eid = jnp.zeros((L,), jnp.int32)
                for f in range(NUM_EXPERTS - 1):
                    eid = eid + jnp.where(blk >= prefix[f], 1, 0)
                if v == 3:
                    eid = jnp.where(iota == 0, nact, 0)
                metav[pl.ds(v * L, L)] = eid
            pltpu.sync_copy(metav, meta_hbm)


def _spread_body(xf_hbm, posp_hbm, xs_hbm,
                 ppa, ppb, pva, pvb, rb0, rb1, psem, sem0, sem1,
                 os0, os1, os2, os3):
    t_tokens = xf_hbm.shape[0]
    wid = lax.axis_index("s") * 2 + lax.axis_index("c")
    rbs = (rb0, rb1)
    pps = (ppa, ppb)
    pvs = (pva, pvb)
    sems = (sem0, sem1)
    osems = ((os0, os1), (os2, os3))

    cps = [None, None]
    for r in range(2):
        g = wid + 32 * r
        for e in range(NUM_EXPERTS):
            pltpu.async_copy(posp_hbm.at[e, pl.ds(g * 32, 32)],
                             pps[r].at[e, pl.ds(0, 32)], psem)
            pltpu.async_copy(posp_hbm.at[e, pl.ds(t_tokens + g * 32, 32)],
                             pps[r].at[e, pl.ds(32, 32)], psem)
        cps[r] = pltpu.async_copy(
            xf_hbm.at[pl.ds(g * 32, 32)], rbs[r], sems[r])

    def drain(j, carry):
        pltpu.make_async_copy(
            posp_hbm.at[0, pl.ds(0, 32)],
            ppa.at[0, pl.ds(0, 32)], psem).wait()
        return carry

    lax.fori_loop(0, 2 * 2 * NUM_EXPERTS, drain, 0)

    for r in range(2):
        for q in range(4):
            acc = pps[r][0, pl.ds(q * L, L)]
            for e in range(1, NUM_EXPERTS):
                acc = jnp.maximum(acc, pps[r][e, pl.ds(q * L, L)])
            pvs[r][q // 2, pl.ds((q % 2) * L, L)] = acc - 1

    ops = []
    for r in range(2):
        cps[r].wait()
        ops.append(pltpu.async_copy(
            rbs[r], xs_hbm.at[pvs[r].at[0]], osems[r][0]))
        ops.append(pltpu.async_copy(
            rbs[r], xs_hbm.at[pvs[r].at[1]], osems[r][1]))
    for o in ops:
        o.wait()


def _mm_body(meta_ref, xs_ref, w1_ref, w2_ref, wv_ref, out_ref):
    b = pl.program_id(0)
    nact = meta_ref[48]

    @pl.when(b < nact)
    def _():
        h = lax.dot_general(
            xs_ref[0], w1_ref[...], (((1,), (0,)), ((), ())),
            preferred_element_type=jnp.float32,
        )
        y = lax.dot_general(
            h, w2_ref[...], (((1,), (0,)), ((), ())),
            preferred_element_type=jnp.float32,
        )
        out_ref[0] = y * wv_ref[0]


def _scatter_body(ys_hbm, slots_hbm, meta_hbm, buf_hbm,
                  sall, sidx, rb0, rb1, metav, sem0, sem1, osem0, osem1):
    nslots = buf_hbm.shape[0] - 256
    t_tokens = nslots // 2
    wid = lax.axis_index("s") * 2 + lax.axis_index("c")
    iota = lax.iota(jnp.int32, L)
    pltpu.sync_copy(meta_hbm.at[pl.ds(48, L)], metav)
    nch = 4 * jnp.max(metav[...])
    pltpu.sync_copy(slots_hbm, sall)
    rbs = (rb0, rb1)
    sems = (sem0, sem1)
    osems = (osem0, osem1)

    for r in range(REPS):
        g = wid + 32 * r
        base = g * CH
        for q in range(CH // L):
            s = sall[pl.ds(base + q * L, L)]
            real = jnp.logical_and(s < nslots, g < nch)
            dst = jnp.where(
                real,
                (s & 1) * t_tokens + lax.shift_right_logical(s, 1),
                nslots + ((base + q * L + iota) & 255))
            sidx[r, pl.ds(q * L, L)] = dst
    cps = [None, None]
    ops = [None, None]
    cps[0] = pltpu.async_copy(ys_hbm.at[pl.ds(wid * CH, CH)], rb0, sem0)
    for r in range(REPS):
        if r < REPS - 1:
            if ops[(r + 1) % 2] is not None:
                ops[(r + 1) % 2].wait()
                ops[(r + 1) % 2] = None
            cps[(r + 1) % 2] = pltpu.async_copy(
                ys_hbm.at[pl.ds((wid + 32 * (r + 1)) * CH, CH)],
                rbs[(r + 1) % 2], sems[(r + 1) % 2])
        cps[r % 2].wait()
        ops[r % 2] = pltpu.async_copy(
            rbs[r % 2], buf_hbm.at[sidx.at[r]], osems[r % 2])
    for q in range(2):
        if ops[q] is not None:
            ops[q].wait()


def _pair_body(a_ref, b_ref, out_ref):
    out_ref[...] = a_ref[...] + b_ref[...]


def kernel(x, w_router, w1, w2):
    b, s, d = x.shape
    t = b * s
    nslots = 2 * t
    xf = x.reshape(t, d)

    sel, wn = pl.pallas_call(
        _router_body,
        out_shape=(
            jax.ShapeDtypeStruct((t, 2), jnp.int32),
            jax.ShapeDtypeStruct((t, 2), jnp.float32),
        ),
    )(xf, w_router)

    sel_flat = sel.reshape(nslots)
    wn_flat = wn.reshape(nslots)

    mesh = plsc.VectorSubcoreMesh(core_axis_name="c", subcore_axis_name="s")
    sc_params = pltpu.CompilerParams(needs_layout_passes=False)

    slots, wvec, meta, posp = pl.kernel(
        _index_body,
        out_type=(
            jax.ShapeDtypeStruct((NR,), jnp.int32),
            jax.ShapeDtypeStruct((NR,), jnp.float32),
            jax.ShapeDtypeStruct((64,), jnp.int32),
            jax.ShapeDtypeStruct((NUM_EXPERTS, nslots), jnp.int32),
        ),
        mesh=mesh,
        scratch_types=[
            pltpu.VMEM((nslots,), jnp.int32),
            pltpu.VMEM((nslots,), jnp.float32),
            pltpu.VMEM((NR,), jnp.int32),
            pltpu.VMEM((NR,), jnp.float32),
            pltpu.VMEM((64,), jnp.int32),
            pltpu.VMEM((nslots,), jnp.int32),
            pltpu.SemaphoreType.DMA,
        ],
        compiler_params=sc_params,
    )(sel_flat, wn_flat)

    xs = pl.kernel(
        _spread_body,
        out_type=jax.ShapeDtypeStruct((NR, d), jnp.float32),
        mesh=mesh,
        scratch_types=[
            pltpu.VMEM((NUM_EXPERTS, 64), jnp.int32),
            pltpu.VMEM((NUM_EXPERTS, 64), jnp.int32),
            pltpu.VMEM((2, 32), jnp.int32),
            pltpu.VMEM((2, 32), jnp.int32),
            pltpu.VMEM((32, d), jnp.float32),
            pltpu.VMEM((32, d), jnp.float32),
            pltpu.SemaphoreType.DMA,
            pltpu.SemaphoreType.DMA,
            pltpu.SemaphoreType.DMA,
            pltpu.SemaphoreType.DMA,
            pltpu.SemaphoreType.DMA,
            pltpu.SemaphoreType.DMA,
            pltpu.SemaphoreType.DMA,
        ],
        compiler_params=sc_params,
    )(xf, posp)

    xs3 = xs.reshape(NB, BLK, d)
    wv3 = wvec.reshape(NB, BLK, 1)

    ys = pl.pallas_call(
        _mm_body,
        grid_spec=pltpu.PrefetchScalarGridSpec(
            num_scalar_prefetch=1,
            grid=(NB,),
            in_specs=[
                pl.BlockSpec((1, BLK, d),
                             lambda i, m: (jnp.minimum(i, m[48] - 1), 0, 0)),
                pl.BlockSpec((d, D_FFN), lambda i, m: (0, m[i])),
                pl.BlockSpec((D_FFN, d), lambda i, m: (m[i], 0)),
                pl.BlockSpec((1, BLK, 1),
                             lambda i, m: (jnp.minimum(i, m[48] - 1), 0, 0)),
            ],
            out_specs=pl.BlockSpec(
                (1, BLK, d), lambda i, m: (jnp.minimum(i, m[48] - 1), 0, 0)),
        ),
        out_shape=jax.ShapeDtypeStruct((NB, BLK, d), jnp.float32),
        compiler_params=pltpu.CompilerParams(
            dimension_semantics=("arbitrary",),
        ),
    )(meta, xs3, w1, w2, wv3)

    buf = pl.kernel(
        _scatter_body,
        out_type=jax.ShapeDtypeStruct((nslots + 256, d), jnp.float32),
        mesh=mesh,
        scratch_types=[
            pltpu.VMEM((NR,), jnp.int32),
            pltpu.VMEM((REPS, CH), jnp.int32),
            pltpu.VMEM((CH, d), jnp.float32),
            pltpu.VMEM((CH, d), jnp.float32),
            pltpu.VMEM((L,), jnp.int32),
            pltpu.SemaphoreType.DMA,
            pltpu.SemaphoreType.DMA,
            pltpu.SemaphoreType.DMA,
            pltpu.SemaphoreType.DMA,
        ],
        compiler_params=sc_params,
    )(ys.reshape(NR, d), slots, meta)

    nt = t // BLK
    out = pl.pallas_call(
        _pair_body,
        grid=(nt,),
        in_specs=[
            pl.BlockSpec((BLK, d), lambda i: (i, 0)),
            pl.BlockSpec((BLK, d), lambda i: (nt + i, 0)),
        ],
        out_specs=pl.BlockSpec((BLK, d), lambda i: (i, 0)),
        out_shape=jax.ShapeDtypeStruct((t, d), jnp.float32),
    )(buf, buf)

    return out.reshape(b, s, d)


# speedup vs baseline: 1.4652x; 1.0572x over previous
import functools

import jax
import jax.numpy as jnp
from jax import lax
from jax.experimental import pallas as pl
from jax.experimental.pallas import tpu as pltpu
from jax.experimental.pallas import tpu_sc as plsc

NUM_EXPERTS = 8
N_EMBD = 1024
D_FFN = 2048
BLK = 128
NB = 40
NR = NB * BLK
L = 16
CH = 32
NCH = NR // CH
REPS = NCH // 32


def _router_body(x_ref, wr_ref, sel_ref, wn_ref):
    x = x_ref[...]
    wr = wr_ref[...]
    logits = lax.dot_general(
        x, wr, (((1,), (1,)), ((), ())), preferred_element_type=jnp.float32
    )
    m = jnp.max(logits, axis=-1, keepdims=True)
    ex = jnp.exp(logits - m)
    probs = ex / jnp.sum(ex, axis=-1, keepdims=True)
    e_iota = lax.broadcasted_iota(jnp.int32, probs.shape, 1)
    m1 = jnp.max(probs, axis=-1, keepdims=True)
    i1 = jnp.min(jnp.where(probs == m1, e_iota, NUM_EXPERTS), axis=-1, keepdims=True)
    masked = jnp.where(e_iota == i1, -jnp.inf, probs)
    m2 = jnp.max(masked, axis=-1, keepdims=True)
    i2 = jnp.min(jnp.where(masked == m2, e_iota, NUM_EXPERTS), axis=-1, keepdims=True)
    s = m1 + m2
    sel_ref[...] = jnp.concatenate([i1, i2], axis=1)
    wn_ref[...] = jnp.concatenate([m1 / s, m2 / s], axis=1)


def _index_body(sel_hbm, wn_hbm, slots_hbm, wvec_hbm, meta_hbm, posp_hbm,
                selv, wnv, slotsv, wvecv, metav, posv, psem):
    nslots = sel_hbm.shape[0]
    nv = nslots // L
    wid = lax.axis_index("s") * 2 + lax.axis_index("c")
    iota = lax.iota(jnp.int32, L)

    @pl.when(wid < NUM_EXPERTS + 1)
    def _():
        pltpu.sync_copy(sel_hbm, selv)

        def count_step(j, acc):
            v = selv[pl.ds(j * L, L)]
            return tuple(
                acc[f] + jnp.where(v == f, 1, 0) for f in range(NUM_EXPERTS)
            )

        zero = jnp.zeros((L,), jnp.int32)
        acc = lax.fori_loop(0, nv, count_step, (zero,) * NUM_EXPERTS)
        counts = [jnp.sum(a) for a in acc]
        bcs = [(c + BLK - 1) // BLK for c in counts]

        @pl.when(wid < NUM_EXPERTS)
        def _():
            pltpu.sync_copy(wn_hbm, wnv)
            e = wid
            base = jnp.int32(0)
            for f in range(NUM_EXPERTS):
                base = base + jnp.where(f < e, bcs[f], 0)
            base = base * BLK
            my_bc = jnp.int32(0)
            for f in range(NUM_EXPERTS):
                my_bc = my_bc + jnp.where(f == e, bcs[f], 0)

            def fill_step(j, carry):
                off = base + j * L
                slotsv[pl.ds(off, L)] = nslots + ((off + iota) & 255)
                wvecv[pl.ds(off, L)] = jnp.zeros((L,), jnp.float32)
                return carry

            lax.fori_loop(0, my_bc * (BLK // L), fill_step, 0)

            def zero_step(j, carry):
                posv[pl.ds(j * L, L)] = jnp.zeros((L,), jnp.int32)
                return carry

            lax.fori_loop(0, nslots // L, zero_step, 0)

            tt = nslots // 2

            def rank_step(j, cnt):
                sl = j * L + iota
                v = selv[pl.ds(j * L, L)]
                mk = v == e
                c = plsc.cumsum(jnp.where(mk, 1, 0))
                p = base + cnt + c - 1
                plsc.store_scatter(slotsv, [p], sl, mask=mk)
                wvals = wnv[pl.ds(j * L, L)]
                plsc.store_scatter(wvecv, [p], wvals, mask=mk)
                td = (sl & 1) * tt + lax.shift_right_logical(sl, 1)
                plsc.store_scatter(posv, [td], p + 1, mask=mk)
                return cnt + jnp.where(mk, 1, 0).sum()

            lax.fori_loop(0, nv, rank_step, jnp.int32(0))

            maxbc = nslots // BLK
            for j in range(maxbc):
                @pl.when(j < my_bc)
                def _(j=j):
                    off = base + j * BLK
                    pltpu.async_copy(slotsv.at[pl.ds(off, BLK)],
                                     slots_hbm.at[pl.ds(off, BLK)], psem)
                    pltpu.async_copy(wvecv.at[pl.ds(off, BLK)],
                                     wvec_hbm.at[pl.ds(off, BLK)], psem)

            pltpu.sync_copy(posv, posp_hbm.at[e])

            def drain(j, carry):
                pltpu.make_async_copy(
                    slots_hbm.at[pl.ds(0, BLK)],
                    slotsv.at[pl.ds(0, BLK)], psem).wait()
                return carry

            lax.fori_loop(0, my_bc * 2, drain, 0)

        @pl.when(wid == NUM_EXPERTS)
        def _():
            prefix = []
            run = jnp.int32(0)
            for f in range(NUM_EXPERTS):
                run = run + bcs[f]
                prefix.append(run)
            nact = prefix[-1]
            for v in range(4):
                blk = v * L + iota
                eid = jnp.zeros((L,), jnp.int32)
                for f in range(NUM_EXPERTS - 1):
                    eid = eid + jnp.where(blk >= prefix[f], 1, 0)
                if v == 3:
                    eid = jnp.where(iota == 0, nact, 0)
                metav[pl.ds(v * L, L)] = eid
            pltpu.sync_copy(metav, meta_hbm)


def _spread_body(xf_hbm, posp_hbm, xs_hbm,
                 ppa, ppb, pva, pvb, rb0, rb1, psem, sem0, sem1,
                 os0, os1, os2, os3):
    t_tokens = xf_hbm.shape[0]
    wid = lax.axis_index("s") * 2 + lax.axis_index("c")
    rbs = (rb0, rb1)
    pps = (ppa, ppb)
    pvs = (pva, pvb)
    sems = (sem0, sem1)
    osems = ((os0, os1), (os2, os3))

    cps = [None, None]
    for r in range(2):
        g = wid + 32 * r
        for e in range(NUM_EXPERTS):
            pltpu.async_copy(posp_hbm.at[e, pl.ds(g * 32, 32)],
                             pps[r].at[e, pl.ds(0, 32)], psem)
            pltpu.async_copy(posp_hbm.at[e, pl.ds(t_tokens + g * 32, 32)],
                             pps[r].at[e, pl.ds(32, 32)], psem)
        cps[r] = pltpu.async_copy(
            xf_hbm.at[pl.ds(g * 32, 32)], rbs[r], sems[r])

    def drain(j, carry):
        pltpu.make_async_copy(
            posp_hbm.at[0, pl.ds(0, 32)],
            ppa.at[0, pl.ds(0, 32)], psem).wait()
        return carry

    lax.fori_loop(0, 2 * 2 * NUM_EXPERTS, drain, 0)

    for r in range(2):
        for q in range(4):
            acc = pps[r][0, pl.ds(q * L, L)]
            for e in range(1, NUM_EXPERTS):
                acc = jnp.maximum(acc, pps[r][e, pl.ds(q * L, L)])
            pvs[r][q // 2, pl.ds((q % 2) * L, L)] = acc - 1

    ops = []
    for r in range(2):
        cps[r].wait()
        ops.append(pltpu.async_copy(
            rbs[r], xs_hbm.at[pvs[r].at[0]], osems[r][0]))
        ops.append(pltpu.async_copy(
            rbs[r], xs_hbm.at[pvs[r].at[1]], osems[r][1]))
    for o in ops:
        o.wait()


def _mm_body(meta_ref, xs_ref, w1_ref, w2_ref, wv_ref, out_ref):
    b = pl.program_id(0)
    nact = meta_ref[48]

    @pl.when(b < nact)
    def _():
        h = lax.dot_general(
            xs_ref[0], w1_ref[...], (((1,), (0,)), ((), ())),
            preferred_element_type=jnp.float32,
        )
        y = lax.dot_general(
            h, w2_ref[...], (((1,), (0,)), ((), ())),
            preferred_element_type=jnp.float32,
        )
        y = y * wv_ref[0]
        dp = y.shape[1] // 2

        def rne16(v):
            bits = lax.bitcast_convert_type(v, jnp.int32)
            rnd = bits + 0x7FFF + (lax.shift_right_logical(bits, 16) & 1)
            return lax.shift_right_logical(rnd, 16)

        lo = rne16(y[:, :dp])
        hi = rne16(y[:, dp:])
        out_ref[0] = lo | lax.shift_left(hi, 16)


def _scatter_body(ys_hbm, slots_hbm, meta_hbm, buf_hbm,
                  sall, sidx, rb0, rb1, metav, sem0, sem1, osem0, osem1):
    nslots = buf_hbm.shape[0] - 256
    t_tokens = nslots // 2
    wid = lax.axis_index("s") * 2 + lax.axis_index("c")
    iota = lax.iota(jnp.int32, L)
    pltpu.sync_copy(meta_hbm.at[pl.ds(48, L)], metav)
    nch = 4 * jnp.max(metav[...])
    pltpu.sync_copy(slots_hbm, sall)
    rbs = (rb0, rb1)
    sems = (sem0, sem1)
    osems = (osem0, osem1)

    for r in range(REPS):
        g = wid + 32 * r
        base = g * CH
        for q in range(CH // L):
            s = sall[pl.ds(base + q * L, L)]
            real = jnp.logical_and(s < nslots, g < nch)
            dst = jnp.where(
                real,
                (s & 1) * t_tokens + lax.shift_right_logical(s, 1),
                nslots + ((base + q * L + iota) & 255))
            sidx[r, pl.ds(q * L, L)] = dst
    cps = [None, None]
    ops = [None, None]
    cps[0] = pltpu.async_copy(ys_hbm.at[pl.ds(wid * CH, CH)], rb0, sem0)
    for r in range(REPS):
        if r < REPS - 1:
            if ops[(r + 1) % 2] is not None:
                ops[(r + 1) % 2].wait()
                ops[(r + 1) % 2] = None
            cps[(r + 1) % 2] = pltpu.async_copy(
                ys_hbm.at[pl.ds((wid + 32 * (r + 1)) * CH, CH)],
                rbs[(r + 1) % 2], sems[(r + 1) % 2])
        cps[r % 2].wait()
        ops[r % 2] = pltpu.async_copy(
            rbs[r % 2], buf_hbm.at[sidx.at[r]], osems[r % 2])
    for q in range(2):
        if ops[q] is not None:
            ops[q].wait()


def _pair_body(a_ref, b_ref, out_ref):
    dp = a_ref.shape[1]

    def unpack(p):
        lo = lax.bitcast_convert_type(lax.shift_left(p, 16), jnp.float32)
        hi = lax.bitcast_convert_type(p & jnp.int32(-65536), jnp.float32)
        return lo, hi

    alo, ahi = unpack(a_ref[...])
    blo, bhi = unpack(b_ref[...])
    out_ref[:, :dp] = alo + blo
    out_ref[:, dp:] = ahi + bhi


def kernel(x, w_router, w1, w2):
    b, s, d = x.shape
    t = b * s
    nslots = 2 * t
    xf = x.reshape(t, d)

    sel, wn = pl.pallas_call(
        _router_body,
        out_shape=(
            jax.ShapeDtypeStruct((t, 2), jnp.int32),
            jax.ShapeDtypeStruct((t, 2), jnp.float32),
        ),
    )(xf, w_router)

    sel_flat = sel.reshape(nslots)
    wn_flat = wn.reshape(nslots)

    mesh = plsc.VectorSubcoreMesh(core_axis_name="c", subcore_axis_name="s")
    sc_params = pltpu.CompilerParams(needs_layout_passes=False)

    slots, wvec, meta, posp = pl.kernel(
        _index_body,
        out_type=(
            jax.ShapeDtypeStruct((NR,), jnp.int32),
            jax.ShapeDtypeStruct((NR,), jnp.float32),
            jax.ShapeDtypeStruct((64,), jnp.int32),
            jax.ShapeDtypeStruct((NUM_EXPERTS, nslots), jnp.int32),
        ),
        mesh=mesh,
        scratch_types=[
            pltpu.VMEM((nslots,), jnp.int32),
            pltpu.VMEM((nslots,), jnp.float32),
            pltpu.VMEM((NR,), jnp.int32),
            pltpu.VMEM((NR,), jnp.float32),
            pltpu.VMEM((64,), jnp.int32),
            pltpu.VMEM((nslots,), jnp.int32),
            pltpu.SemaphoreType.DMA,
        ],
        compiler_params=sc_params,
    )(sel_flat, wn_flat)

    xs = pl.kernel(
        _spread_body,
        out_type=jax.ShapeDtypeStruct((NR, d), jnp.float32),
        mesh=mesh,
        scratch_types=[
            pltpu.VMEM((NUM_EXPERTS, 64), jnp.int32),
            pltpu.VMEM((NUM_EXPERTS, 64), jnp.int32),
            pltpu.VMEM((2, 32), jnp.int32),
            pltpu.VMEM((2, 32), jnp.int32),
            pltpu.VMEM((32, d), jnp.float32),
            pltpu.VMEM((32, d), jnp.float32),
            pltpu.SemaphoreType.DMA,
            pltpu.SemaphoreType.DMA,
            pltpu.SemaphoreType.DMA,
            pltpu.SemaphoreType.DMA,
            pltpu.SemaphoreType.DMA,
            pltpu.SemaphoreType.DMA,
            pltpu.SemaphoreType.DMA,
        ],
        compiler_params=sc_params,
    )(xf, posp)

    xs3 = xs.reshape(NB, BLK, d)
    wv3 = wvec.reshape(NB, BLK, 1)

    ys = pl.pallas_call(
        _mm_body,
        grid_spec=pltpu.PrefetchScalarGridSpec(
            num_scalar_prefetch=1,
            grid=(NB,),
            in_specs=[
                pl.BlockSpec((1, BLK, d),
                             lambda i, m: (jnp.minimum(i, m[48] - 1), 0, 0)),
                pl.BlockSpec((d, D_FFN), lambda i, m: (0, m[i])),
                pl.BlockSpec((D_FFN, d), lambda i, m: (m[i], 0)),
                pl.BlockSpec((1, BLK, 1),
                             lambda i, m: (jnp.minimum(i, m[48] - 1), 0, 0)),
            ],
            out_specs=pl.BlockSpec(
                (1, BLK, d // 2),
                lambda i, m: (jnp.minimum(i, m[48] - 1), 0, 0)),
        ),
        out_shape=jax.ShapeDtypeStruct((NB, BLK, d // 2), jnp.int32),
        compiler_params=pltpu.CompilerParams(
            dimension_semantics=("arbitrary",),
        ),
    )(meta, xs3, w1, w2, wv3)

    buf = pl.kernel(
        _scatter_body,
        out_type=jax.ShapeDtypeStruct((nslots + 256, d // 2), jnp.int32),
        mesh=mesh,
        scratch_types=[
            pltpu.VMEM((NR,), jnp.int32),
            pltpu.VMEM((REPS, CH), jnp.int32),
            pltpu.VMEM((CH, d // 2), jnp.int32),
            pltpu.VMEM((CH, d // 2), jnp.int32),
            pltpu.VMEM((L,), jnp.int32),
            pltpu.SemaphoreType.DMA,
            pltpu.SemaphoreType.DMA,
            pltpu.SemaphoreType.DMA,
            pltpu.SemaphoreType.DMA,
        ],
        compiler_params=sc_params,
    )(ys.reshape(NR, d // 2), slots, meta)

    nt = t // BLK
    out = pl.pallas_call(
        _pair_body,
        grid=(nt,),
        in_specs=[
            pl.BlockSpec((BLK, d // 2), lambda i: (i, 0)),
            pl.BlockSpec((BLK, d // 2), lambda i: (nt + i, 0)),
        ],
        out_specs=pl.BlockSpec((BLK, d), lambda i: (i, 0)),
        out_shape=jax.ShapeDtypeStruct((t, d), jnp.float32),
    )(buf, buf)

    return out.reshape(b, s, d)
